# baseline (speedup 1.0000x reference)
import functools

import jax
import jax.numpy as jnp
import numpy as np
from jax import lax
from jax.experimental import pallas as pl
from jax.experimental.pallas import tpu as pltpu
from jax.experimental.pallas import tpu_sc as plsc

VOCAB = 1000000
EMBED = 64
NUM_CLASS = 4
BATCH = 16384
HIST = 50
EPS = 1e-5

NC, NS, LANES = 2, 16, 16
NW = NC * NS
BPW = BATCH // NW
CHUNK = 16
NCHUNKS = BPW // CHUNK
KCOL = EMBED // LANES
CIDX = CHUNK * HIST

_sc_mesh = plsc.VectorSubcoreMesh(core_axis_name="c", subcore_axis_name="s")


@functools.partial(
    pl.kernel,
    out_type=jax.ShapeDtypeStruct((BATCH, EMBED), jnp.float32),
    mesh=_sc_mesh,
    scratch_types=[
        pltpu.VMEM((CHUNK, 128), jnp.int32),
        pltpu.VMEM((CHUNK, 128), jnp.int32),
        pltpu.VMEM((CIDX,), jnp.int32),
        pltpu.VMEM((CIDX,), jnp.int32),
        pltpu.VMEM((CIDX, 128), jnp.bfloat16),
        pltpu.VMEM((CIDX, 128), jnp.bfloat16),
        pltpu.VMEM((CHUNK, EMBED), jnp.float32),
        pltpu.VMEM((CHUNK, EMBED), jnp.float32),
        pltpu.SemaphoreType.DMA,
    ],
    compiler_params=pltpu.CompilerParams(use_tc_tiling_on_sc=False,
                                         needs_layout_passes=False),
)
def _bag_sums(x_hbm, table_hbm, out_hbm,
              idxr0, idxr1, idxf0, idxf1, rows0, rows1, out0, out1, gsem):
    wid = lax.axis_index("s") * NC + lax.axis_index("c")
    idxr = (idxr0, idxr1)
    idxf = (idxf0, idxf1)
    rows = (rows0, rows1)
    outs = (out0, out1)
    lane = lax.iota(jnp.int32, 16)

    def fire(ci, slot):
        base = wid * BPW + ci * CHUNK
        pltpu.sync_copy(x_hbm.at[pl.ds(base, CHUNK)], idxr[slot])
        for b in range(CHUNK):
            for off in (0, 16, 32):
                v = idxr[slot][b, pl.ds(off, 16)]
                plsc.store_scatter(idxf[slot], [lane + (b * HIST + off)], v)
            v = idxr[slot][b, pl.ds(48, 16)]
            plsc.store_scatter(idxf[slot], [lane + (b * HIST + 48)], v,
                               mask=lane < (HIST - 48))
        pltpu.async_copy(table_hbm.at[idxf[slot]], rows[slot], gsem)

    def drain_reduce_store(ci, slot):
        base = wid * BPW + ci * CHUNK
        pltpu.make_async_copy(table_hbm.at[idxf[slot]], rows[slot],
                              gsem).wait()
        rv = rows[slot]
        ov = outs[slot]

        def bag_body(b, carry):
            def row_accs(r):
                h0 = rv[b * HIST + r, pl.ds(0, 2 * LANES)]
                h1 = rv[b * HIST + r, pl.ds(2 * LANES, 2 * LANES)]
                a0, b0 = plsc.unpack(h0, format=plsc.PackFormat.INTERLEAVED)
                a1, b1 = plsc.unpack(h1, format=plsc.PackFormat.INTERLEAVED)
                return [a0, b0, a1, b1]

            accs = row_accs(0)
            for r in range(1, HIST):
                nxt = row_accs(r)
                for k in range(KCOL):
                    accs[k] = accs[k] + nxt[k]
            for k in range(KCOL):
                ov[b, pl.ds(k * LANES, LANES)] = accs[k]
            return carry

        lax.fori_loop(0, CHUNK, bag_body, 0)
        pltpu.sync_copy(ov, out_hbm.at[pl.ds(base, CHUNK)])

    fire(0, 0)

    def pair_body(i, carry):
        c0 = 2 * i
        fire(c0 + 1, 1)
        drain_reduce_store(c0, 0)

        @pl.when(c0 + 2 < NCHUNKS)
        def _():
            fire(c0 + 2, 0)

        drain_reduce_store(c0 + 1, 1)
        return carry

    lax.fori_loop(0, NCHUNKS // 2, pair_body, 0)


def _mlp_body(bag_ref, s1_ref, b1_ref, w1_ref, s2_ref, b2_ref, w2_ref,
              fc2b_ref, out_ref):
    h = bag_ref[...] * s1_ref[...] + b1_ref[...]
    h = jnp.maximum(h, 0.0)
    h = jnp.dot(h, w1_ref[...], preferred_element_type=jnp.float32)
    h = h * s2_ref[...] + b2_ref[...]
    h = jnp.maximum(h, 0.0)
    out_ref[...] = (
        jnp.dot(h, w2_ref[...], preferred_element_type=jnp.float32)
        + fc2b_ref[...]
    )


_BM = 2048


def _mlp(sums, s1, b1, w1, s2, b2, w2, fc2b):
    grid = (BATCH // _BM,)
    return pl.pallas_call(
        _mlp_body,
        grid=grid,
        in_specs=[
            pl.BlockSpec((_BM, EMBED), lambda i: (i, 0)),
            pl.BlockSpec((1, EMBED), lambda i: (0, 0)),
            pl.BlockSpec((1, EMBED), lambda i: (0, 0)),
            pl.BlockSpec((EMBED, 128), lambda i: (0, 0)),
            pl.BlockSpec((1, 128), lambda i: (0, 0)),
            pl.BlockSpec((1, 128), lambda i: (0, 0)),
            pl.BlockSpec((128, NUM_CLASS), lambda i: (0, 0)),
            pl.BlockSpec((1, NUM_CLASS), lambda i: (0, 0)),
        ],
        out_specs=pl.BlockSpec((_BM, NUM_CLASS), lambda i: (i, 0)),
        out_shape=jax.ShapeDtypeStruct((BATCH, NUM_CLASS), jnp.float32),
    )(sums, s1, b1, w1, s2, b2, w2, fc2b)


def kernel(x, emb_table, fc1_w, fc1_b, fc2_w, fc2_b,
           bn1_gamma, bn1_beta, bn2_gamma, bn2_beta):
    xp = jnp.pad(x.astype(jnp.int32), ((0, 0), (0, 128 - HIST)))
    tb = jnp.pad(emb_table.astype(jnp.bfloat16), ((0, 0), (0, 128 - EMBED)))
    sums = _bag_sums(xp, tb)
    perm = np.concatenate([np.arange(0, 32, 2), np.arange(1, 32, 2),
                           np.arange(32, 64, 2), np.arange(33, 64, 2)])
    inv = 1.0 / jnp.sqrt(1.0 + EPS)
    s1 = (bn1_gamma * inv / HIST)[perm].reshape(1, EMBED)
    b1 = bn1_beta[perm].reshape(1, EMBED)
    s2 = (bn2_gamma * inv).reshape(1, 128)
    b2 = (fc1_b * bn2_gamma * inv + bn2_beta).reshape(1, 128)
    return _mlp(sums, s1, b1, fc1_w.T[perm], s2, b2, fc2_w.T,
                fc2_b.reshape(1, NUM_CLASS))


# speedup vs baseline: 1.8736x; 1.8736x over previous
import functools

import jax
import jax.numpy as jnp
from jax import lax
from jax.experimental import pallas as pl
from jax.experimental.pallas import tpu as pltpu
from jax.experimental.pallas import tpu_sc as plsc

VOCAB = 1000000
EMBED = 64
NUM_CLASS = 4
BATCH = 16384
HIST = 50
EPS = 1e-5

NC, NS, LANES = 2, 16, 16
NW = NC * NS
BPW = BATCH // NW
CHUNK = 16
NCHUNKS = BPW // CHUNK
KCOL = EMBED // LANES
CIDX = CHUNK * HIST

_sc_mesh = plsc.VectorSubcoreMesh(core_axis_name="c", subcore_axis_name="s")


@functools.partial(
    pl.kernel,
    out_type=jax.ShapeDtypeStruct((BATCH, EMBED), jnp.float32),
    mesh=_sc_mesh,
    scratch_types=[
        pltpu.VMEM((CHUNK, 128), jnp.int32),
        pltpu.VMEM((CHUNK, 128), jnp.int32),
        pltpu.VMEM((CIDX,), jnp.int32),
        pltpu.VMEM((CIDX,), jnp.int32),
        pltpu.VMEM((CIDX, EMBED), jnp.float32),
        pltpu.VMEM((CIDX, EMBED), jnp.float32),
        pltpu.VMEM((CHUNK, EMBED), jnp.float32),
        pltpu.VMEM((CHUNK, EMBED), jnp.float32),
        pltpu.SemaphoreType.DMA,
    ],
    compiler_params=pltpu.CompilerParams(use_tc_tiling_on_sc=False,
                                         needs_layout_passes=False),
)
def _bag_sums(x_hbm, table_hbm, out_hbm,
              idxr0, idxr1, idxf0, idxf1, rows0, rows1, out0, out1, gsem):
    wid = lax.axis_index("s") * NC + lax.axis_index("c")
    idxr = (idxr0, idxr1)
    idxf = (idxf0, idxf1)
    rows = (rows0, rows1)
    outs = (out0, out1)
    lane = lax.iota(jnp.int32, 16)

    def fire(ci, slot):
        base = wid * BPW + ci * CHUNK
        pltpu.sync_copy(x_hbm.at[pl.ds(base, CHUNK)], idxr[slot])
        for b in range(CHUNK):
            for off in (0, 16, 32):
                v = idxr[slot][b, pl.ds(off, 16)]
                plsc.store_scatter(idxf[slot], [lane + (b * HIST + off)], v)
            v = idxr[slot][b, pl.ds(48, 16)]
            plsc.store_scatter(idxf[slot], [lane + (b * HIST + 48)], v,
                               mask=lane < (HIST - 48))
        pltpu.async_copy(table_hbm.at[idxf[slot]], rows[slot], gsem)

    def drain_reduce_store(ci, slot):
        base = wid * BPW + ci * CHUNK
        pltpu.make_async_copy(table_hbm.at[idxf[slot]], rows[slot],
                              gsem).wait()
        rv = rows[slot]
        ov = outs[slot]

        def bag_body(b, carry):
            accs = [rv[b * HIST, pl.ds(k * LANES, LANES)] for k in range(KCOL)]
            for r in range(1, HIST):
                for k in range(KCOL):
                    accs[k] = accs[k] + rv[b * HIST + r, pl.ds(k * LANES, LANES)]
            for k in range(KCOL):
                ov[b, pl.ds(k * LANES, LANES)] = accs[k]
            return carry

        lax.fori_loop(0, CHUNK, bag_body, 0)
        pltpu.sync_copy(ov, out_hbm.at[pl.ds(base, CHUNK)])

    fire(0, 0)

    def pair_body(i, carry):
        c0 = 2 * i
        fire(c0 + 1, 1)
        drain_reduce_store(c0, 0)

        @pl.when(c0 + 2 < NCHUNKS)
        def _():
            fire(c0 + 2, 0)

        drain_reduce_store(c0 + 1, 1)
        return carry

    lax.fori_loop(0, NCHUNKS // 2, pair_body, 0)


def _mlp_body(bag_ref, s1_ref, b1_ref, w1_ref, s2_ref, b2_ref, w2_ref,
              fc2b_ref, out_ref):
    h = bag_ref[...] * s1_ref[...] + b1_ref[...]
    h = jnp.maximum(h, 0.0)
    h = jnp.dot(h, w1_ref[...], preferred_element_type=jnp.float32)
    h = h * s2_ref[...] + b2_ref[...]
    h = jnp.maximum(h, 0.0)
    out_ref[...] = (
        jnp.dot(h, w2_ref[...], preferred_element_type=jnp.float32)
        + fc2b_ref[...]
    )


_BM = 2048


def _mlp(sums, s1, b1, w1, s2, b2, w2, fc2b):
    grid = (BATCH // _BM,)
    return pl.pallas_call(
        _mlp_body,
        grid=grid,
        in_specs=[
            pl.BlockSpec((_BM, EMBED), lambda i: (i, 0)),
            pl.BlockSpec((1, EMBED), lambda i: (0, 0)),
            pl.BlockSpec((1, EMBED), lambda i: (0, 0)),
            pl.BlockSpec((EMBED, 128), lambda i: (0, 0)),
            pl.BlockSpec((1, 128), lambda i: (0, 0)),
            pl.BlockSpec((1, 128), lambda i: (0, 0)),
            pl.BlockSpec((128, NUM_CLASS), lambda i: (0, 0)),
            pl.BlockSpec((1, NUM_CLASS), lambda i: (0, 0)),
        ],
        out_specs=pl.BlockSpec((_BM, NUM_CLASS), lambda i: (i, 0)),
        out_shape=jax.ShapeDtypeStruct((BATCH, NUM_CLASS), jnp.float32),
    )(sums, s1, b1, w1, s2, b2, w2, fc2b)


def kernel(x, emb_table, fc1_w, fc1_b, fc2_w, fc2_b,
           bn1_gamma, bn1_beta, bn2_gamma, bn2_beta):
    xp = jnp.pad(x.astype(jnp.int32), ((0, 0), (0, 128 - HIST)))
    sums = _bag_sums(xp, emb_table)
    inv = 1.0 / jnp.sqrt(1.0 + EPS)
    s1 = (bn1_gamma * inv / HIST).reshape(1, EMBED)
    b1 = bn1_beta.reshape(1, EMBED)
    s2 = (bn2_gamma * inv).reshape(1, 128)
    b2 = (fc1_b * bn2_gamma * inv + bn2_beta).reshape(1, 128)
    return _mlp(sums, s1, b1, fc1_w.T, s2, b2, fc2_w.T,
                fc2_b.reshape(1, NUM_CLASS))
